# Initial kernel scaffold; baseline (speedup 1.0000x reference)
import jax
import jax.numpy as jnp
from jax.experimental import pallas as pl


def kernel(x, edge_index, edge_weight, Wl0, Wr0, bl0, Wl1, Wr1, bl1, Wl2, Wr2, bl2, Wout, bout):
    raise NotImplementedError("write your pallas kernel here")


# speedup vs baseline: 2.4176x; 2.4176x over previous
import functools

import jax
import jax.numpy as jnp
from jax import lax
from jax.experimental import pallas as pl
from jax.experimental.pallas import tpu as pltpu
from jax.experimental.pallas import tpu_sc as plsc

N = 10000
E = 160000
C = 128
NTILE = 16
EPT = E // NTILE
WIN = 80
SUB = 80
NSUB = WIN // SUB
NWIN = EPT // WIN
RPT = 640
NPAD = 10240
DPT = NPAD // NTILE
DH = 512
DOUT = 256


def _mesh():
    return plsc.VectorSubcoreMesh(core_axis_name="c", subcore_axis_name="s")


@functools.lru_cache(maxsize=None)
def _prop(nch, weighted):
    nch2 = nch // 2

    def body(*refs):
        if weighted:
            (src, cidx1, ridx1, ew, zeros, out,
             acc, buf, cidx_v, ew_v, *ridx_vs) = refs
        else:
            (src, cidx1, ridx1, zeros, out,
             acc, buf, cidx_v, *ridx_vs) = refs
        c = lax.axis_index("c")
        s = lax.axis_index("s")
        for kl in range(nch2):
            kk = c * nch2 + kl if nch2 > 1 else c
            pltpu.sync_copy(zeros, acc.at[pl.ds(s * RPT, RPT)])
            plsc.subcore_barrier()

            @pl.loop(0, NWIN)
            def _(w):
                base = s * EPT + w * WIN
                pltpu.sync_copy(cidx1.at[pl.ds(base, WIN)], cidx_v)
                for j in range(NSUB):
                    pltpu.sync_copy(ridx1.at[pl.ds(base + j * SUB, SUB)],
                                    ridx_vs[j])
                for j in range(NSUB):
                    pltpu.sync_copy(src.at[kk].at[cidx_v.at[pl.ds(j * SUB, SUB)]],
                                    buf.at[pl.ds(j * SUB, SUB)])
                if weighted:
                    pltpu.sync_copy(ew.at[pl.ds(base, WIN)], ew_v)

                    @pl.loop(0, WIN, step=16)
                    def _(e0):
                        wvec = ew_v[pl.ds(e0, 16)]
                        for l in range(16):
                            we = wvec[l]
                            for j in range(C // 16):
                                sl = (e0 + l, pl.ds(j * 16, 16))
                                buf[sl] = buf[sl] * we
                for j in range(NSUB):
                    pltpu.sync_copy(buf.at[pl.ds(j * SUB, SUB)],
                                    acc.at[ridx_vs[j]], add=True)

            plsc.subcore_barrier()
            pltpu.sync_copy(acc.at[pl.ds(s * RPT, RPT)],
                            out.at[kk].at[pl.ds(s * RPT, RPT)])
            if kl + 1 < nch2:
                plsc.subcore_barrier()

    scratch = [
        pltpu.VMEM_SHARED((NPAD, C), jnp.float32),
        pltpu.VMEM((WIN, C), jnp.float32),
        pltpu.VMEM((WIN,), jnp.int32),
    ]
    if weighted:
        scratch.append(pltpu.VMEM((WIN,), jnp.float32))
    scratch.extend(pltpu.VMEM((SUB,), jnp.int32) for _ in range(NSUB))
    return pl.kernel(
        body,
        out_type=jax.ShapeDtypeStruct((nch, NPAD, C), jnp.float32),
        mesh=_mesh(),
        scratch_types=scratch,
    )


@functools.lru_cache(maxsize=None)
def _deg():
    def body(ridx1, zeros1, out, acc1, ones_v, val_v, *ridx_vs):
        s = lax.axis_index("s")
        pltpu.sync_copy(zeros1, acc1.at[pl.ds(s * DPT, DPT)])
        for off in sorted({*range(0, SUB - 15, 16), SUB - 16}):
            ones_v[pl.ds(off, 16)] = jnp.full((16,), 1.0, jnp.float32)
        plsc.subcore_barrier()

        @pl.loop(0, NWIN)
        def _(w):
            base = s * EPT + w * WIN
            for j in range(NSUB):
                pltpu.sync_copy(ridx1.at[pl.ds(base + j * SUB, SUB)],
                                ridx_vs[j])
            for j in range(NSUB):
                pltpu.sync_copy(ones_v, acc1.at[ridx_vs[j]], add=True)

        plsc.subcore_barrier()
        pltpu.sync_copy(acc1.at[pl.ds(s * DPT, DPT)], val_v)
        for j in range(DPT // 16):
            v = val_v[pl.ds(j * 16, 16)]
            val_v[pl.ds(j * 16, 16)] = 1.0 / jnp.maximum(v, 1.0)
        pltpu.sync_copy(val_v, out.at[pl.ds(s * DPT, DPT)])

    return pl.kernel(
        body,
        out_type=jax.ShapeDtypeStruct((NPAD,), jnp.float32),
        mesh=_mesh(),
        scratch_types=(
            [pltpu.VMEM_SHARED((NPAD,), jnp.float32),
             pltpu.VMEM((SUB,), jnp.float32),
             pltpu.VMEM((DPT,), jnp.float32)]
            + [pltpu.VMEM((SUB,), jnp.int32) for _ in range(NSUB)]
        ),
    )


def _ewnorm(ew2):
    def body(a_ref, o_ref):
        m = jnp.max(a_ref[...])
        o_ref[...] = a_ref[...] / (m + 1e-6)

    return pl.pallas_call(
        body,
        out_shape=jax.ShapeDtypeStruct((E // 128, 128), jnp.float32),
    )(ew2)


_R = 512


@functools.lru_cache(maxsize=None)
def _tc_layer(nchin):
    din = nchin * C

    def body(nb_ref, g_ref, idg_ref, wl_ref, wr_ref, bl_ref, out_ref):
        dn = (((1,), (1,)), ((), ()))
        accl = jnp.zeros((_R, DH), jnp.float32)
        accr = jnp.zeros((_R, DH), jnp.float32)
        for k in range(nchin):
            accl += lax.dot_general(nb_ref[k], wl_ref[:, k * C:(k + 1) * C],
                                    dn, preferred_element_type=jnp.float32)
            accr += lax.dot_general(g_ref[k], wr_ref[:, k * C:(k + 1) * C],
                                    dn, preferred_element_type=jnp.float32)
        h = jnp.maximum(accl * idg_ref[...] + accr + bl_ref[...], 0.0)
        for ko in range(DH // C):
            out_ref[ko] = h[:, ko * C:(ko + 1) * C]

    return pl.pallas_call(
        body,
        grid=(NPAD // _R,),
        in_specs=[
            pl.BlockSpec((nchin, _R, C), lambda i: (0, i, 0)),
            pl.BlockSpec((nchin, _R, C), lambda i: (0, i, 0)),
            pl.BlockSpec((_R, 1), lambda i: (i, 0)),
            pl.BlockSpec((DH, din), lambda i: (0, 0)),
            pl.BlockSpec((DH, din), lambda i: (0, 0)),
            pl.BlockSpec((1, DH), lambda i: (0, 0)),
        ],
        out_specs=pl.BlockSpec((DH // C, _R, C), lambda i: (0, i, 0)),
        out_shape=jax.ShapeDtypeStruct((DH // C, NPAD, C), jnp.float32),
    )


@functools.lru_cache(maxsize=None)
def _tc_out():
    def body(h_ref, wo_ref, bo_ref, out_ref):
        dn = (((1,), (1,)), ((), ()))
        acc = jnp.zeros((_R, DOUT), jnp.float32)
        for k in range(DH // C):
            acc += lax.dot_general(h_ref[k], wo_ref[:, k * C:(k + 1) * C],
                                   dn, preferred_element_type=jnp.float32)
        out_ref[...] = acc + bo_ref[...]

    return pl.pallas_call(
        body,
        grid=(NPAD // _R,),
        in_specs=[
            pl.BlockSpec((DH // C, _R, C), lambda i: (0, i, 0)),
            pl.BlockSpec((DOUT, DH), lambda i: (0, 0)),
            pl.BlockSpec((1, DOUT), lambda i: (0, 0)),
        ],
        out_specs=pl.BlockSpec((_R, DOUT), lambda i: (i, 0)),
        out_shape=jax.ShapeDtypeStruct((NPAD, DOUT), jnp.float32),
    )


def kernel(x, edge_index, edge_weight, Wl0, Wr0, bl0, Wl1, Wr1, bl1,
           Wl2, Wr2, bl2, Wout, bout):
    row = edge_index[0]
    col = edge_index[1]
    ewn = _ewnorm(edge_weight.reshape(E // 128, 128)).reshape(E)
    zeros2 = jnp.zeros((RPT, C), jnp.float32)
    zeros1 = jnp.zeros((DPT,), jnp.float32)
    idg = _deg()(row, zeros1).reshape(NPAD, 1)
    x2 = jnp.zeros((2, NPAD, C), jnp.float32).at[:, :N].set(
        x.reshape(N, 2, C).transpose(1, 0, 2))

    g = _prop(2, True)(x2, col, row, ewn, zeros2)
    nb = _prop(2, False)(g, col, row, zeros2)
    h = _tc_layer(2)(nb, g, idg, Wl0, Wr0, bl0.reshape(1, DH))
    for Wl, Wr, bl in ((Wl1, Wr1, bl1), (Wl2, Wr2, bl2)):
        g = _prop(4, True)(h, col, row, ewn, zeros2)
        nb = _prop(4, False)(g, col, row, zeros2)
        h = _tc_layer(4)(nb, g, idg, Wl, Wr, bl.reshape(1, DH))
    return _tc_out()(h, Wout, bout.reshape(1, DOUT))[:N]


# speedup vs baseline: 5.4293x; 2.2458x over previous
import functools

import jax
import jax.numpy as jnp
from jax import lax
from jax.experimental import pallas as pl
from jax.experimental.pallas import tpu as pltpu
from jax.experimental.pallas import tpu_sc as plsc

N = 10000
E = 160000
NPAD = 10240
EP = 163840
C = 128
NTILE = 16
SUB = 128
GRP = 8
NGRP = (EP // NTILE) // (SUB * GRP)
RPT = NPAD // NTILE
DPT = NPAD // NTILE
DH = 512
DOUT = 256


def _mesh():
    return plsc.VectorSubcoreMesh(core_axis_name="c", subcore_axis_name="s")


@functools.lru_cache(maxsize=None)
def _prop(nch, weighted):
    nch2 = nch // 2

    def body(*refs):
        if weighted:
            (src, cidx2, ridx2, ew2, zeros, out, acc,
             buf0, buf1, cidxg, ridxg, ewg, g0, g1, s0, s1) = refs
        else:
            (src, cidx2, ridx2, zeros, out, acc,
             buf0, buf1, cidxg, ridxg, g0, g1, s0, s1) = refs
        bufs = (buf0, buf1)
        gsem = (g0, g1)
        ssem = (s0, s1)
        c = lax.axis_index("c")
        s = lax.axis_index("s")
        for kl in range(nch2):
            kk = c * nch2 + kl if nch2 > 1 else c
            pltpu.sync_copy(zeros, acc.at[pl.ds(s * RPT, RPT)])
            plsc.subcore_barrier()

            @pl.loop(0, NGRP)
            def _(g):
                rbase = s * (NGRP * GRP) + g * GRP
                pltpu.sync_copy(cidx2.at[pl.ds(rbase, GRP)], cidxg)
                pltpu.sync_copy(ridx2.at[pl.ds(rbase, GRP)], ridxg)
                if weighted:
                    pltpu.sync_copy(ew2.at[pl.ds(rbase, GRP)], ewg)
                gd = {}
                sd = {}
                gd[0] = pltpu.async_copy(src.at[kk].at[cidxg.at[0]],
                                         bufs[0], gsem[0])
                for j in range(GRP):
                    p = j & 1
                    if j + 1 < GRP:
                        if j >= 1:
                            sd[1 - p].wait()
                        gd[j + 1] = pltpu.async_copy(
                            src.at[kk].at[cidxg.at[j + 1]],
                            bufs[1 - p], gsem[1 - p])
                    gd[j].wait()
                    if weighted:
                        buf = bufs[p]

                        @pl.loop(0, SUB, step=16)
                        def _(e0):
                            wvec = ewg[j, pl.ds(e0, 16)]
                            for l in range(16):
                                we = wvec[l]
                                for q in range(C // 16):
                                    sl = (e0 + l, pl.ds(q * 16, 16))
                                    buf[sl] = buf[sl] * we
                    sd[p] = pltpu.async_copy(bufs[p], acc.at[ridxg.at[j]],
                                             ssem[p], add=True)
                sd[0].wait()
                sd[1].wait()

            plsc.subcore_barrier()
            pltpu.sync_copy(acc.at[pl.ds(s * RPT, RPT)],
                            out.at[kk].at[pl.ds(s * RPT, RPT)])
            if kl + 1 < nch2:
                plsc.subcore_barrier()

    scratch = [
        pltpu.VMEM_SHARED((NPAD, C), jnp.float32),
        pltpu.VMEM((SUB, C), jnp.float32),
        pltpu.VMEM((SUB, C), jnp.float32),
        pltpu.VMEM((GRP, SUB), jnp.int32),
        pltpu.VMEM((GRP, SUB), jnp.int32),
    ]
    if weighted:
        scratch.append(pltpu.VMEM((GRP, SUB), jnp.float32))
    scratch.extend([pltpu.SemaphoreType.DMA] * 4)
    return pl.kernel(
        body,
        out_type=jax.ShapeDtypeStruct((nch, NPAD, C), jnp.float32),
        mesh=_mesh(),
        scratch_types=scratch,
    )


@functools.lru_cache(maxsize=None)
def _deg():
    def body(ridx2, zeros1, out, acc1, ones_v, val_v, ridxg):
        s = lax.axis_index("s")
        pltpu.sync_copy(zeros1, acc1.at[pl.ds(s * DPT, DPT)])
        for off in range(0, SUB, 16):
            ones_v[pl.ds(off, 16)] = jnp.full((16,), 1.0, jnp.float32)
        plsc.subcore_barrier()

        @pl.loop(0, NGRP)
        def _(g):
            rbase = s * (NGRP * GRP) + g * GRP
            pltpu.sync_copy(ridx2.at[pl.ds(rbase, GRP)], ridxg)
            for j in range(GRP):
                pltpu.sync_copy(ones_v, acc1.at[ridxg.at[j]], add=True)

        plsc.subcore_barrier()
        pltpu.sync_copy(acc1.at[pl.ds(s * DPT, DPT)], val_v)
        for j in range(DPT // 16):
            v = val_v[pl.ds(j * 16, 16)]
            val_v[pl.ds(j * 16, 16)] = 1.0 / jnp.maximum(v, 1.0)
        pltpu.sync_copy(val_v, out.at[pl.ds(s * DPT, DPT)])

    return pl.kernel(
        body,
        out_type=jax.ShapeDtypeStruct((NPAD,), jnp.float32),
        mesh=_mesh(),
        scratch_types=[
            pltpu.VMEM_SHARED((NPAD,), jnp.float32),
            pltpu.VMEM((SUB,), jnp.float32),
            pltpu.VMEM((DPT,), jnp.float32),
            pltpu.VMEM((GRP, SUB), jnp.int32),
        ],
    )


def _ewnorm(ew2):
    def body(a_ref, o_ref):
        m = jnp.max(a_ref[...])
        o_ref[...] = a_ref[...] / (m + 1e-6)

    return pl.pallas_call(
        body,
        out_shape=jax.ShapeDtypeStruct((E // 128, 128), jnp.float32),
    )(ew2)


_R = 512


@functools.lru_cache(maxsize=None)
def _tc_layer(nchin):
    din = nchin * C

    def body(nb_ref, g_ref, idg_ref, wl_ref, wr_ref, bl_ref, out_ref):
        dn = (((1,), (1,)), ((), ()))
        accl = jnp.zeros((_R, DH), jnp.float32)
        accr = jnp.zeros((_R, DH), jnp.float32)
        for k in range(nchin):
            accl += lax.dot_general(nb_ref[k], wl_ref[:, k * C:(k + 1) * C],
                                    dn, preferred_element_type=jnp.float32)
            accr += lax.dot_general(g_ref[k], wr_ref[:, k * C:(k + 1) * C],
                                    dn, preferred_element_type=jnp.float32)
        h = jnp.maximum(accl * idg_ref[...] + accr + bl_ref[...], 0.0)
        for ko in range(DH // C):
            out_ref[ko] = h[:, ko * C:(ko + 1) * C]

    return pl.pallas_call(
        body,
        grid=(NPAD // _R,),
        in_specs=[
            pl.BlockSpec((nchin, _R, C), lambda i: (0, i, 0)),
            pl.BlockSpec((nchin, _R, C), lambda i: (0, i, 0)),
            pl.BlockSpec((_R, 1), lambda i: (i, 0)),
            pl.BlockSpec((DH, din), lambda i: (0, 0)),
            pl.BlockSpec((DH, din), lambda i: (0, 0)),
            pl.BlockSpec((1, DH), lambda i: (0, 0)),
        ],
        out_specs=pl.BlockSpec((DH // C, _R, C), lambda i: (0, i, 0)),
        out_shape=jax.ShapeDtypeStruct((DH // C, NPAD, C), jnp.float32),
    )


@functools.lru_cache(maxsize=None)
def _tc_out():
    def body(h_ref, wo_ref, bo_ref, out_ref):
        dn = (((1,), (1,)), ((), ()))
        acc = jnp.zeros((_R, DOUT), jnp.float32)
        for k in range(DH // C):
            acc += lax.dot_general(h_ref[k], wo_ref[:, k * C:(k + 1) * C],
                                   dn, preferred_element_type=jnp.float32)
        out_ref[...] = acc + bo_ref[...]

    return pl.pallas_call(
        body,
        grid=(NPAD // _R,),
        in_specs=[
            pl.BlockSpec((DH // C, _R, C), lambda i: (0, i, 0)),
            pl.BlockSpec((DOUT, DH), lambda i: (0, 0)),
            pl.BlockSpec((1, DOUT), lambda i: (0, 0)),
        ],
        out_specs=pl.BlockSpec((_R, DOUT), lambda i: (i, 0)),
        out_shape=jax.ShapeDtypeStruct((NPAD, DOUT), jnp.float32),
    )


def kernel(x, edge_index, edge_weight, Wl0, Wr0, bl0, Wl1, Wr1, bl1,
           Wl2, Wr2, bl2, Wout, bout):
    row = edge_index[0]
    col = edge_index[1]
    ewn = _ewnorm(edge_weight.reshape(E // 128, 128)).reshape(E)
    padidx = (jnp.arange(EP - E, dtype=jnp.int32) % (NPAD - N)) + N
    ridx2 = jnp.concatenate([row, padidx]).reshape(EP // 128, 128)
    cidx2 = jnp.concatenate([col, padidx]).reshape(EP // 128, 128)
    ew2 = jnp.concatenate(
        [ewn, jnp.zeros((EP - E,), jnp.float32)]).reshape(EP // 128, 128)
    zeros2 = jnp.zeros((RPT, C), jnp.float32)
    zeros1 = jnp.zeros((DPT,), jnp.float32)
    idg = _deg()(ridx2, zeros1).reshape(NPAD, 1)
    x2 = jnp.zeros((2, NPAD, C), jnp.float32).at[:, :N].set(
        x.reshape(N, 2, C).transpose(1, 0, 2))

    g = _prop(2, True)(x2, cidx2, ridx2, ew2, zeros2)
    nb = _prop(2, False)(g, cidx2, ridx2, zeros2)
    h = _tc_layer(2)(nb, g, idg, Wl0, Wr0, bl0.reshape(1, DH))
    for Wl, Wr, bl in ((Wl1, Wr1, bl1), (Wl2, Wr2, bl2)):
        g = _prop(4, True)(h, cidx2, ridx2, ew2, zeros2)
        nb = _prop(4, False)(g, cidx2, ridx2, zeros2)
        h = _tc_layer(4)(nb, g, idg, Wl, Wr, bl.reshape(1, DH))
    return _tc_out()(h, Wout, bout.reshape(1, DOUT))[:N]
